# baseline (speedup 1.0000x reference)
import functools

import jax
import jax.numpy as jnp
from jax import lax
from jax.experimental import pallas as pl
from jax.experimental.pallas import tpu as pltpu
from jax.experimental.pallas import tpu_sc as plsc

EMBED_D = 64
BATCH_N = 16384

_NC = 2
_NS = 16
_NW = _NC * _NS
_B_PER_W = BATCH_N // _NW
_CHUNK = 128
_NCHUNK = _B_PER_W // _CHUNK


def _make_gather():
    mesh = plsc.VectorSubcoreMesh(core_axis_name="c", subcore_axis_name="s")

    @functools.partial(
        pl.kernel,
        mesh=mesh,
        out_type=jax.ShapeDtypeStruct((_NW, _NCHUNK, _CHUNK, EMBED_D), jnp.float32),
        scratch_types=[
            pltpu.VMEM((_NCHUNK, _CHUNK), jnp.int32),
            pltpu.VMEM((_NCHUNK, _CHUNK, EMBED_D), jnp.float32),
            pltpu.SemaphoreType.DMA,
        ],
        compiler_params=pltpu.CompilerParams(use_tc_tiling_on_sc=False),
    )
    def k(idx_hbm, table_hbm, out_hbm, idx_v, rows_v, sem):
        wid = lax.axis_index("s") * _NC + lax.axis_index("c")
        pltpu.sync_copy(idx_hbm.at[wid], idx_v)
        copies = [
            pltpu.async_copy(table_hbm.at[idx_v.at[j]], rows_v.at[j], sem)
            for j in range(_NCHUNK)
        ]
        for c in copies:
            c.wait()
        pltpu.sync_copy(rows_v, out_hbm.at[wid])

    return k


_gather = _make_gather()


def kernel(entities, table):
    idx = entities.astype(jnp.int32).reshape(_NW, _NCHUNK, _CHUNK)
    out = _gather(idx, table)
    return out.reshape(BATCH_N, EMBED_D)


# speedup vs baseline: 1.7158x; 1.7158x over previous
import functools

import jax
import jax.numpy as jnp
from jax import lax
from jax.experimental import pallas as pl
from jax.experimental.pallas import tpu as pltpu
from jax.experimental.pallas import tpu_sc as plsc

EMBED_D = 64
BATCH_N = 16384

_NC = 2
_NS = 16
_NW = _NC * _NS
_B_PER_W = BATCH_N // _NW


def _make_gather():
    mesh = plsc.VectorSubcoreMesh(core_axis_name="c", subcore_axis_name="s")

    @functools.partial(
        pl.kernel,
        mesh=mesh,
        out_type=jax.ShapeDtypeStruct((_NW, _B_PER_W, EMBED_D), jnp.float32),
        scratch_types=[
            pltpu.VMEM((_B_PER_W,), jnp.int32),
            pltpu.VMEM((_B_PER_W, EMBED_D), jnp.float32),
            pltpu.SemaphoreType.DMA,
            pltpu.SemaphoreType.DMA,
        ],
        compiler_params=pltpu.CompilerParams(
            use_tc_tiling_on_sc=True, needs_layout_passes=False
        ),
    )
    def k(idx_hbm, table_hbm, out_hbm, idx_v, rows_v, sem_in, sem_out):
        wid = lax.axis_index("s") * _NC + lax.axis_index("c")
        pltpu.sync_copy(idx_hbm.at[wid], idx_v)

        def body(t, carry):
            base = t * 16
            ev = idx_v[pl.ds(base, 16)]
            for l in range(16):
                pltpu.async_copy(
                    table_hbm.at[ev[l]], rows_v.at[base + l], sem_in
                )
            return carry

        lax.fori_loop(0, _B_PER_W // 16, body, 0)
        pltpu.make_async_copy(out_hbm.at[wid], rows_v, sem_in).wait()
        pltpu.async_copy(rows_v, out_hbm.at[wid], sem_out).wait()

    return k


_gather = _make_gather()


def kernel(entities, table):
    idx = entities.astype(jnp.int32).reshape(_NW, _B_PER_W)
    out = _gather(idx, table)
    return out.reshape(BATCH_N, EMBED_D)
